# baseline (speedup 1.0000x reference)
import functools

import jax
import jax.numpy as jnp
from jax import lax
from jax.experimental import pallas as pl
from jax.experimental.pallas import tpu as pltpu
from jax.experimental.pallas import tpu_sc as plsc

B = 16384
NSPLIT = 2
B2 = B // NSPLIT
GMF_DIM = 32
MLP_DIM = 128

NC = 2
NS = 16
NW = NC * NS
CH = 128

_sc_mesh = plsc.VectorSubcoreMesh(
    core_axis_name="c", subcore_axis_name="s", num_cores=NC, num_subcores=NS
)


def _make_mlp_gather(nb):
    bpw = nb // NW
    nch = bpw // CH

    @functools.partial(
        pl.kernel,
        out_type=(
            jax.ShapeDtypeStruct((nb, MLP_DIM), jnp.float32),
            jax.ShapeDtypeStruct((nb, MLP_DIM), jnp.float32),
        ),
        mesh=_sc_mesh,
        scratch_types=(
            pltpu.VMEM((nch, CH), jnp.int32),
            pltpu.VMEM((nch, CH), jnp.int32),
            pltpu.VMEM((CH, MLP_DIM), jnp.float32),
            pltpu.VMEM((CH, MLP_DIM), jnp.float32),
            pltpu.SemaphoreType.DMA,
        ),
    )
    def gather_mlp(ui_hbm, ii_hbm, mlp_p_hbm, mlp_q_hbm,
                   mu_out, mq_out,
                   ui_v, ii_v, mu_v, mq_v, sem):
        wid = lax.axis_index("s") * NC + lax.axis_index("c")
        base = wid * bpw
        for c in range(nch):
            pltpu.sync_copy(ui_hbm.at[pl.ds(base + c * CH, CH)], ui_v.at[c])
            pltpu.sync_copy(ii_hbm.at[pl.ds(base + c * CH, CH)], ii_v.at[c])
        for c in range(nch):
            cp1 = pltpu.async_copy(mlp_p_hbm.at[ui_v.at[c]], mu_v, sem)
            cp2 = pltpu.async_copy(mlp_q_hbm.at[ii_v.at[c]], mq_v, sem)
            cp1.wait()
            cp2.wait()
            off = base + c * CH
            pltpu.sync_copy(mu_v, mu_out.at[pl.ds(off, CH)])
            pltpu.sync_copy(mq_v, mq_out.at[pl.ds(off, CH)])

    return gather_mlp


_GH = 4


def _make_gmf_gather(nb):
    bpw = nb // NW
    nh = bpw // _GH

    @functools.partial(
        pl.kernel,
        out_type=(
            jax.ShapeDtypeStruct((nb, GMF_DIM), jnp.float32),
            jax.ShapeDtypeStruct((nb, GMF_DIM), jnp.float32),
        ),
        mesh=_sc_mesh,
        scratch_types=(
            pltpu.VMEM((bpw + 16,), jnp.int32),
            pltpu.VMEM((bpw + 16,), jnp.int32),
            pltpu.VMEM((_GH, GMF_DIM, 128), jnp.float32),
            pltpu.VMEM((_GH, GMF_DIM, 128), jnp.float32),
            pltpu.VMEM((_GH, GMF_DIM, 128), jnp.float32),
            pltpu.VMEM((_GH, GMF_DIM, 128), jnp.float32),
            pltpu.VMEM((2 * _GH, GMF_DIM), jnp.float32),
            pltpu.VMEM((2 * _GH, GMF_DIM), jnp.float32),
            pltpu.SemaphoreType.DMA,
            pltpu.SemaphoreType.DMA,
        ),
        compiler_params=pltpu.CompilerParams(disable_bounds_checks=True,
                                             needs_layout_passes=False),
    )
    def gather_gmf(ui_hbm, ii_hbm, pt_hbm, qt_hbm,
                   gu_out, gi_out,
                   ui_s, ii_s, pb_a, qb_a, pb_b, qb_b, gu_v, gi_v,
                   sem_a, sem_b):
        wid = lax.axis_index("s") * NC + lax.axis_index("c")
        base = wid * bpw
        pltpu.sync_copy(ui_hbm.at[pl.ds(base, bpw)], ui_s.at[pl.ds(0, bpw)])
        pltpu.sync_copy(ii_hbm.at[pl.ds(base, bpw)], ii_s.at[pl.ds(0, bpw)])
        dlo = lax.iota(jnp.int32, 16)

        def fire(uvec, ivec, lane0, pb, qb, sem):
            for k in range(_GH):
                u = uvec[lane0 + k]
                i = ivec[lane0 + k]
                cu = pl.multiple_of((u // 128) * 128, 128)
                ci = pl.multiple_of((i // 128) * 128, 128)
                pltpu.async_copy(pt_hbm.at[:, pl.ds(cu, 128)], pb.at[k], sem)
                pltpu.async_copy(qt_hbm.at[:, pl.ds(ci, 128)], qb.at[k], sem)

        def drain(pb, qb, sem):
            for k in range(_GH):
                pltpu.make_async_copy(pt_hbm.at[:, pl.ds(0, 128)], pb.at[k], sem).wait()
                pltpu.make_async_copy(qt_hbm.at[:, pl.ds(0, 128)], qb.at[k], sem).wait()

        def select(uvec, ivec, lane0, row0, pb, qb):
            ucol = uvec - (uvec // 128) * 128
            icol = ivec - (ivec // 128) * 128
            for k in range(_GH):
                colu = jnp.full((16,), ucol[lane0 + k], jnp.int32)
                coli = jnp.full((16,), icol[lane0 + k], jnp.int32)
                r = row0 + k
                gu_v[r, pl.ds(0, 16)] = plsc.load_gather(pb.at[k], [dlo, colu])
                gu_v[r, pl.ds(16, 16)] = plsc.load_gather(pb.at[k], [dlo + 16, colu])
                gi_v[r, pl.ds(0, 16)] = plsc.load_gather(qb.at[k], [dlo, coli])
                gi_v[r, pl.ds(16, 16)] = plsc.load_gather(qb.at[k], [dlo + 16, coli])

        uv0 = ui_s[pl.ds(0, 16)]
        iv0 = ii_s[pl.ds(0, 16)]
        fire(uv0, iv0, 0, pb_a, qb_a, sem_a)

        def body(m, carry):
            j0 = m * 2 * _GH
            uvec = ui_s[pl.ds(j0, 16)]
            ivec = ii_s[pl.ds(j0, 16)]
            uvec_n = ui_s[pl.ds(j0 + 2 * _GH, 16)]
            ivec_n = ii_s[pl.ds(j0 + 2 * _GH, 16)]
            fire(uvec, ivec, _GH, pb_b, qb_b, sem_b)
            drain(pb_a, qb_a, sem_a)
            select(uvec, ivec, 0, 0, pb_a, qb_a)
            fire(uvec_n, ivec_n, 0, pb_a, qb_a, sem_a)
            drain(pb_b, qb_b, sem_b)
            select(uvec, ivec, _GH, _GH, pb_b, qb_b)
            pltpu.sync_copy(gu_v, gu_out.at[pl.ds(base + j0, 2 * _GH)])
            pltpu.sync_copy(gi_v, gi_out.at[pl.ds(base + j0, 2 * _GH)])
            return carry

        lax.fori_loop(0, nh // 2 - 1, body, 0)

        j0 = (nh - 2) * _GH
        uvec = ui_s[pl.ds(j0, 16)]
        ivec = ii_s[pl.ds(j0, 16)]
        fire(uvec, ivec, _GH, pb_b, qb_b, sem_b)
        drain(pb_a, qb_a, sem_a)
        select(uvec, ivec, 0, 0, pb_a, qb_a)
        drain(pb_b, qb_b, sem_b)
        select(uvec, ivec, _GH, _GH, pb_b, qb_b)
        pltpu.sync_copy(gu_v, gu_out.at[pl.ds(base + j0, 2 * _GH)])
        pltpu.sync_copy(gi_v, gi_out.at[pl.ds(base + j0, 2 * _GH)])

    return gather_gmf


_BB = 1024


def _mlp_body(gu_ref, gi_ref, mu_ref, mq_ref,
              w1_ref, b1_ref, w2_ref, b2_ref, w3_ref, b3_ref,
              wo_ref, bo_ref, out_ref):
    dg = lax.dot_general
    f32 = jnp.float32
    h = dg(mu_ref[...], w1_ref[0:MLP_DIM, :], (((1,), (0,)), ((), ())),
           preferred_element_type=f32)
    h = h + dg(mq_ref[...], w1_ref[MLP_DIM:2 * MLP_DIM, :], (((1,), (0,)), ((), ())),
               preferred_element_type=f32)
    h = jnp.maximum(h + b1_ref[...], 0.0)
    h = jnp.maximum(dg(h, w2_ref[...], (((1,), (0,)), ((), ())),
                       preferred_element_type=f32) + b2_ref[...], 0.0)
    h = jnp.maximum(dg(h, w3_ref[...], (((1,), (0,)), ((), ())),
                       preferred_element_type=f32) + b3_ref[...], 0.0)
    out = dg(wo_ref[GMF_DIM:2 * GMF_DIM, :], h, (((0,), (1,)), ((), ())),
             preferred_element_type=f32)
    out = out + dg(wo_ref[0:GMF_DIM, :], gu_ref[...] * gi_ref[...],
                   (((0,), (1,)), ((), ())), preferred_element_type=f32)
    out_ref[...] = out + bo_ref[...]


def _mlp_tc(gu, gi, mu, mq, w1, b1, w2, b2, w3, b3, wo, bo):
    nb = gu.shape[0]
    grid = nb // _BB
    return pl.pallas_call(
        _mlp_body,
        grid=(grid,),
        in_specs=[
            pl.BlockSpec((_BB, GMF_DIM), lambda i: (i, 0)),
            pl.BlockSpec((_BB, GMF_DIM), lambda i: (i, 0)),
            pl.BlockSpec((_BB, MLP_DIM), lambda i: (i, 0)),
            pl.BlockSpec((_BB, MLP_DIM), lambda i: (i, 0)),
            pl.BlockSpec((256, 128), lambda i: (0, 0)),
            pl.BlockSpec((1, 128), lambda i: (0, 0)),
            pl.BlockSpec((128, 64), lambda i: (0, 0)),
            pl.BlockSpec((1, 64), lambda i: (0, 0)),
            pl.BlockSpec((64, 32), lambda i: (0, 0)),
            pl.BlockSpec((1, 32), lambda i: (0, 0)),
            pl.BlockSpec((64, 1), lambda i: (0, 0)),
            pl.BlockSpec((1, 1), lambda i: (0, 0)),
        ],
        out_specs=pl.BlockSpec((1, _BB), lambda i: (0, i)),
        out_shape=jax.ShapeDtypeStruct((1, nb), jnp.float32),
    )(gu, gi, mu, mq, w1, b1, w2, b2, w3, b3, wo, bo)


_gather_mlp_sc = _make_mlp_gather(B2)
_gather_gmf_sc = _make_gmf_gather(B2)


def kernel(user_id, item_id, gmf_P, gmf_Q, mlp_P, mlp_Q,
           W1, b1, W2, b2, W3, b3, Wout, bout):
    ui = user_id - 1
    ii = item_id - 1
    b1r = b1.reshape(1, -1)
    b2r = b2.reshape(1, -1)
    b3r = b3.reshape(1, -1)
    bor = bout.reshape(1, 1)
    pt = gmf_P.T
    qt = gmf_Q.T
    outs = []
    for s in range(NSPLIT):
        uis = lax.dynamic_slice_in_dim(ui, s * B2, B2)
        iis = lax.dynamic_slice_in_dim(ii, s * B2, B2)
        mu, mq = _gather_mlp_sc(uis, iis, mlp_P, mlp_Q)
        gu, gi = _gather_gmf_sc(uis, iis, pt, qt)
        outs.append(_mlp_tc(gu, gi, mu, mq, W1, b1r, W2, b2r, W3, b3r,
                            Wout, bor))
    return jnp.concatenate(outs, axis=1).reshape(B, 1)


# speedup vs baseline: 1.0119x; 1.0119x over previous
import functools

import jax
import jax.numpy as jnp
from jax import lax
from jax.experimental import pallas as pl
from jax.experimental.pallas import tpu as pltpu
from jax.experimental.pallas import tpu_sc as plsc

B = 16384
U = 1000000
GMF_DIM = 32
MLP_DIM = 128

NC = 2
NS = 16
NW = NC * NS
BPW = B // NW
CH = 128
NCH = BPW // CH

_sc_mesh = plsc.VectorSubcoreMesh(
    core_axis_name="c", subcore_axis_name="s", num_cores=NC, num_subcores=NS
)


@functools.partial(
    pl.kernel,
    out_type=(
        jax.ShapeDtypeStruct((B, MLP_DIM), jnp.float32),
        jax.ShapeDtypeStruct((B, MLP_DIM), jnp.float32),
    ),
    mesh=_sc_mesh,
    scratch_types=(
        pltpu.VMEM((NCH, CH), jnp.int32),
        pltpu.VMEM((NCH, CH), jnp.int32),
        pltpu.VMEM((CH, MLP_DIM), jnp.float32),
        pltpu.VMEM((CH, MLP_DIM), jnp.float32),
        pltpu.SemaphoreType.DMA,
    ),
)
def _gather_mlp_sc(ui_hbm, ii_hbm, mlp_p_hbm, mlp_q_hbm,
                   mu_out, mq_out,
                   ui_v, ii_v, mu_v, mq_v, sem):
    wid = lax.axis_index("s") * NC + lax.axis_index("c")
    base = wid * BPW
    for c in range(NCH):
        pltpu.sync_copy(ui_hbm.at[pl.ds(base + c * CH, CH)], ui_v.at[c])
        pltpu.sync_copy(ii_hbm.at[pl.ds(base + c * CH, CH)], ii_v.at[c])
    for c in range(NCH):
        cp1 = pltpu.async_copy(mlp_p_hbm.at[ui_v.at[c]], mu_v, sem)
        cp2 = pltpu.async_copy(mlp_q_hbm.at[ii_v.at[c]], mq_v, sem)
        cp1.wait()
        cp2.wait()
        off = base + c * CH
        pltpu.sync_copy(mu_v, mu_out.at[pl.ds(off, CH)])
        pltpu.sync_copy(mq_v, mq_out.at[pl.ds(off, CH)])


_GH = 4
_NH = BPW // _GH


@functools.partial(
    pl.kernel,
    out_type=(
        jax.ShapeDtypeStruct((B, GMF_DIM), jnp.float32),
        jax.ShapeDtypeStruct((B, GMF_DIM), jnp.float32),
    ),
    mesh=_sc_mesh,
    scratch_types=(
        pltpu.VMEM((BPW + 16,), jnp.int32),
        pltpu.VMEM((BPW + 16,), jnp.int32),
        pltpu.VMEM((_GH, GMF_DIM, 128), jnp.float32),
        pltpu.VMEM((_GH, GMF_DIM, 128), jnp.float32),
        pltpu.VMEM((_GH, GMF_DIM, 128), jnp.float32),
        pltpu.VMEM((_GH, GMF_DIM, 128), jnp.float32),
        pltpu.VMEM((2 * _GH, GMF_DIM), jnp.float32),
        pltpu.VMEM((2 * _GH, GMF_DIM), jnp.float32),
        pltpu.SemaphoreType.DMA,
        pltpu.SemaphoreType.DMA,
    ),
    compiler_params=pltpu.CompilerParams(disable_bounds_checks=True,
                                         needs_layout_passes=False),
)
def _gather_gmf_sc(ui_hbm, ii_hbm, pt_hbm, qt_hbm,
                   gu_out, gi_out,
                   ui_s, ii_s, pb_a, qb_a, pb_b, qb_b, gu_v, gi_v,
                   sem_a, sem_b):
    wid = lax.axis_index("s") * NC + lax.axis_index("c")
    base = wid * BPW
    pltpu.sync_copy(ui_hbm.at[pl.ds(base, BPW)], ui_s.at[pl.ds(0, BPW)])
    pltpu.sync_copy(ii_hbm.at[pl.ds(base, BPW)], ii_s.at[pl.ds(0, BPW)])
    dlo = lax.iota(jnp.int32, 16)

    def fire(uvec, ivec, lane0, pb, qb, sem):
        for k in range(_GH):
            u = uvec[lane0 + k]
            i = ivec[lane0 + k]
            cu = pl.multiple_of((u // 128) * 128, 128)
            ci = pl.multiple_of((i // 128) * 128, 128)
            pltpu.async_copy(pt_hbm.at[:, pl.ds(cu, 128)], pb.at[k], sem)
            pltpu.async_copy(qt_hbm.at[:, pl.ds(ci, 128)], qb.at[k], sem)

    def drain(pb, qb, sem):
        for k in range(_GH):
            pltpu.make_async_copy(pt_hbm.at[:, pl.ds(0, 128)], pb.at[k], sem).wait()
            pltpu.make_async_copy(qt_hbm.at[:, pl.ds(0, 128)], qb.at[k], sem).wait()

    def select(uvec, ivec, lane0, row0, pb, qb):
        ucol = uvec - (uvec // 128) * 128
        icol = ivec - (ivec // 128) * 128
        for k in range(_GH):
            colu = jnp.full((16,), ucol[lane0 + k], jnp.int32)
            coli = jnp.full((16,), icol[lane0 + k], jnp.int32)
            r = row0 + k
            gu_v[r, pl.ds(0, 16)] = plsc.load_gather(pb.at[k], [dlo, colu])
            gu_v[r, pl.ds(16, 16)] = plsc.load_gather(pb.at[k], [dlo + 16, colu])
            gi_v[r, pl.ds(0, 16)] = plsc.load_gather(qb.at[k], [dlo, coli])
            gi_v[r, pl.ds(16, 16)] = plsc.load_gather(qb.at[k], [dlo + 16, coli])

    uv0 = ui_s[pl.ds(0, 16)]
    iv0 = ii_s[pl.ds(0, 16)]
    fire(uv0, iv0, 0, pb_a, qb_a, sem_a)

    def body(m, carry):
        j0 = m * 2 * _GH
        uvec = ui_s[pl.ds(j0, 16)]
        ivec = ii_s[pl.ds(j0, 16)]
        uvec_n = ui_s[pl.ds(j0 + 2 * _GH, 16)]
        ivec_n = ii_s[pl.ds(j0 + 2 * _GH, 16)]
        fire(uvec, ivec, _GH, pb_b, qb_b, sem_b)
        drain(pb_a, qb_a, sem_a)
        select(uvec, ivec, 0, 0, pb_a, qb_a)
        fire(uvec_n, ivec_n, 0, pb_a, qb_a, sem_a)
        drain(pb_b, qb_b, sem_b)
        select(uvec, ivec, _GH, _GH, pb_b, qb_b)
        pltpu.sync_copy(gu_v, gu_out.at[pl.ds(base + j0, 2 * _GH)])
        pltpu.sync_copy(gi_v, gi_out.at[pl.ds(base + j0, 2 * _GH)])
        return carry

    lax.fori_loop(0, _NH // 2 - 1, body, 0)

    j0 = (_NH - 2) * _GH
    uvec = ui_s[pl.ds(j0, 16)]
    ivec = ii_s[pl.ds(j0, 16)]
    fire(uvec, ivec, _GH, pb_b, qb_b, sem_b)
    drain(pb_a, qb_a, sem_a)
    select(uvec, ivec, 0, 0, pb_a, qb_a)
    drain(pb_b, qb_b, sem_b)
    select(uvec, ivec, _GH, _GH, pb_b, qb_b)
    pltpu.sync_copy(gu_v, gu_out.at[pl.ds(base + j0, 2 * _GH)])
    pltpu.sync_copy(gi_v, gi_out.at[pl.ds(base + j0, 2 * _GH)])


_BB = 1024


def _mlp_body(gu_ref, gi_ref, mu_ref, mq_ref,
              w1_ref, b1_ref, w2_ref, b2_ref, w3_ref, b3_ref,
              wo_ref, bo_ref, out_ref):
    dg = lax.dot_general
    f32 = jnp.float32
    h = dg(mu_ref[...], w1_ref[0:MLP_DIM, :], (((1,), (0,)), ((), ())),
           preferred_element_type=f32)
    h = h + dg(mq_ref[...], w1_ref[MLP_DIM:2 * MLP_DIM, :], (((1,), (0,)), ((), ())),
               preferred_element_type=f32)
    h = jnp.maximum(h + b1_ref[...], 0.0)
    h = jnp.maximum(dg(h, w2_ref[...], (((1,), (0,)), ((), ())),
                       preferred_element_type=f32) + b2_ref[...], 0.0)
    h = jnp.maximum(dg(h, w3_ref[...], (((1,), (0,)), ((), ())),
                       preferred_element_type=f32) + b3_ref[...], 0.0)
    out = dg(wo_ref[GMF_DIM:2 * GMF_DIM, :], h, (((0,), (1,)), ((), ())),
             preferred_element_type=f32)
    out = out + dg(wo_ref[0:GMF_DIM, :], gu_ref[...] * gi_ref[...],
                   (((0,), (1,)), ((), ())), preferred_element_type=f32)
    out_ref[...] = out + bo_ref[...]


def _mlp_tc(gu, gi, mu, mq, w1, b1, w2, b2, w3, b3, wo, bo):
    grid = B // _BB
    return pl.pallas_call(
        _mlp_body,
        grid=(grid,),
        in_specs=[
            pl.BlockSpec((_BB, GMF_DIM), lambda i: (i, 0)),
            pl.BlockSpec((_BB, GMF_DIM), lambda i: (i, 0)),
            pl.BlockSpec((_BB, MLP_DIM), lambda i: (i, 0)),
            pl.BlockSpec((_BB, MLP_DIM), lambda i: (i, 0)),
            pl.BlockSpec((256, 128), lambda i: (0, 0)),
            pl.BlockSpec((1, 128), lambda i: (0, 0)),
            pl.BlockSpec((128, 64), lambda i: (0, 0)),
            pl.BlockSpec((1, 64), lambda i: (0, 0)),
            pl.BlockSpec((64, 32), lambda i: (0, 0)),
            pl.BlockSpec((1, 32), lambda i: (0, 0)),
            pl.BlockSpec((64, 1), lambda i: (0, 0)),
            pl.BlockSpec((1, 1), lambda i: (0, 0)),
        ],
        out_specs=pl.BlockSpec((1, _BB), lambda i: (0, i)),
        out_shape=jax.ShapeDtypeStruct((1, B), jnp.float32),
    )(gu, gi, mu, mq, w1, b1, w2, b2, w3, b3, wo, bo)


def kernel(user_id, item_id, gmf_P, gmf_Q, mlp_P, mlp_Q,
           W1, b1, W2, b2, W3, b3, Wout, bout):
    ui = user_id - 1
    ii = item_id - 1
    mu, mq = _gather_mlp_sc(ui, ii, mlp_P, mlp_Q)
    gu, gi = _gather_gmf_sc(ui, ii, gmf_P.T, gmf_Q.T)
    out_t = _mlp_tc(gu, gi, mu, mq,
                    W1, b1.reshape(1, -1), W2, b2.reshape(1, -1),
                    W3, b3.reshape(1, -1), Wout, bout.reshape(1, 1))
    return out_t.reshape(B, 1)
